# Initial kernel scaffold; baseline (speedup 1.0000x reference)
import jax
import jax.numpy as jnp
from jax.experimental import pallas as pl


def kernel(d, p, n, sdf_W1, sdf_b1, sdf_W2, sdf_b2, c1w0, c1a0, c1w1, c1a1, c2w0, c2a0, c2w1, c2a1, c3w0, c3a0, c4w, c4a, l1w, l1a, l2w, l2a, l3w, l3a, l4w, l4a):
    raise NotImplementedError("write your pallas kernel here")


# speedup vs baseline: 16.0618x; 16.0618x over previous
import functools

import jax
import jax.numpy as jnp
from jax import lax
from jax.experimental import pallas as pl
from jax.experimental.pallas import tpu as pltpu
from jax.experimental.pallas import tpu_sc as plsc

K = 10
F = 64
R_KNN = 256
R_MLP = 512
NW = 32
CH = 128


def _prelu(v, a):
    return jnp.where(v >= 0, v, a * v)


def _head(d, p, nrm, W1, b1, W2, b2):
    B, n, _ = d.shape
    W1T = W1.T
    W2T = W2.T

    def body(d_ref, p_ref, n_ref, w1_ref, b1_ref, w2_ref, w2t_ref, w1t_ref,
             b2_ref, x_ref, res_ref, grad_ref, li_ref):
        dd = d_ref[0]
        pp = p_ref[0]
        nn = n_ref[0]
        x = pp + dd * nn
        h = jnp.tanh(jnp.dot(x, w1_ref[...], preferred_element_type=jnp.float32)
                     + b1_ref[...][None, :])
        res = jnp.dot(h, w2_ref[...], preferred_element_type=jnp.float32) + b2_ref[0]
        gp = (1.0 - h * h) * w2t_ref[...]
        grad = jnp.dot(gp, w1t_ref[...], preferred_element_type=jnp.float32)
        x_ref[0] = x
        res_ref[0] = res
        grad_ref[0] = grad
        li_ref[0] = jnp.concatenate([x, nn, res, grad], axis=1)

    return pl.pallas_call(
        body,
        grid=(B,),
        in_specs=[
            pl.BlockSpec((1, n, 1), lambda b: (b, 0, 0)),
            pl.BlockSpec((1, 1, 3), lambda b: (b, 0, 0)),
            pl.BlockSpec((1, n, 3), lambda b: (b, 0, 0)),
            pl.BlockSpec((3, F), lambda b: (0, 0)),
            pl.BlockSpec((F,), lambda b: (0,)),
            pl.BlockSpec((F, 1), lambda b: (0, 0)),
            pl.BlockSpec((1, F), lambda b: (0, 0)),
            pl.BlockSpec((F, 3), lambda b: (0, 0)),
            pl.BlockSpec(memory_space=pltpu.SMEM),
        ],
        out_specs=[
            pl.BlockSpec((1, n, 3), lambda b: (b, 0, 0)),
            pl.BlockSpec((1, n, 1), lambda b: (b, 0, 0)),
            pl.BlockSpec((1, n, 3), lambda b: (b, 0, 0)),
            pl.BlockSpec((1, n, 10), lambda b: (b, 0, 0)),
        ],
        out_shape=[
            jax.ShapeDtypeStruct((B, n, 3), jnp.float32),
            jax.ShapeDtypeStruct((B, n, 1), jnp.float32),
            jax.ShapeDtypeStruct((B, n, 3), jnp.float32),
            jax.ShapeDtypeStruct((B, n, 10), jnp.float32),
        ],
    )(d, p, nrm, W1, b1, W2, W2T, W1T, b2)


def _knn_tab(x3d):
    B, n, c = x3d.shape
    xT = jnp.swapaxes(x3d, 1, 2)
    NB = n // R_KNN

    def body(xf_ref, xt_ref, xb_ref, idx_ref, tab_ref):
        b = pl.program_id(0)
        xf = xf_ref[0]
        xt = xt_ref[0]
        sq_all = jnp.sum(xf * xf, axis=1, keepdims=True)
        sq_r = jnp.sum(xt * xt, axis=0, keepdims=True)
        prod = lax.dot_general(xf, xt, (((1,), (0,)), ((), ())),
                               preferred_element_type=jnp.float32)
        d2 = sq_all + sq_r - 2.0 * prod
        iota = lax.broadcasted_iota(jnp.int32, (n, R_KNN), 0)
        base = b * n
        for t in range(K):
            m = jnp.min(d2, axis=0, keepdims=True)
            amin = jnp.min(jnp.where(d2 == m, iota, n), axis=0, keepdims=True)
            idx_ref[pl.ds(t, 1), :] = amin + base
            d2 = jnp.where(iota == amin, jnp.inf, d2)
        xb = xb_ref[0]
        tab_ref[...] = jnp.concatenate(
            [xb, jnp.zeros((R_KNN, 2 * F - c), jnp.float32)], axis=1)

    return pl.pallas_call(
        body,
        grid=(B, NB),
        in_specs=[
            pl.BlockSpec((1, n, c), lambda b, r: (b, 0, 0)),
            pl.BlockSpec((1, c, R_KNN), lambda b, r: (b, 0, r)),
            pl.BlockSpec((1, R_KNN, c), lambda b, r: (b, r, 0)),
        ],
        out_specs=[
            pl.BlockSpec((K, R_KNN), lambda b, r: (0, b * NB + r)),
            pl.BlockSpec((R_KNN, 2 * F), lambda b, r: (b * NB + r, 0)),
        ],
        out_shape=[
            jax.ShapeDtypeStruct((K, B * n), jnp.int32),
            jax.ShapeDtypeStruct((B * n, 2 * F), jnp.float32),
        ],
    )(x3d, xT, x3d)


def _sc_gather(table, idx):
    nrows = idx.shape[0]
    width = table.shape[1]
    per_w = nrows // NW
    nch = per_w // CH
    mesh = plsc.VectorSubcoreMesh(core_axis_name="c", subcore_axis_name="s")

    @functools.partial(
        pl.kernel,
        out_type=jax.ShapeDtypeStruct((nrows, width), jnp.float32),
        mesh=mesh,
        scratch_types=[
            pltpu.VMEM((CH,), jnp.int32),
            pltpu.VMEM((CH, width), jnp.float32),
            pltpu.SemaphoreType.DMA,
        ],
    )
    def k(table_hbm, idx_hbm, out_hbm, idx_v, rows_v, sem):
        wid = lax.axis_index("s") * 2 + lax.axis_index("c")
        base = wid * per_w
        for ci in range(nch):
            off = base + ci * CH
            pltpu.sync_copy(idx_hbm.at[pl.ds(off, CH)], idx_v)
            pltpu.async_copy(table_hbm.at[idx_v], rows_v, sem).wait()
            pltpu.sync_copy(rows_v, out_hbm.at[pl.ds(off, CH)])

    return k(table, idx)


def _edge_mlp(G3, Xc, W0, W1, alphas):
    BN, c = Xc.shape
    NB = BN // R_MLP
    has_w1 = W1 is not None

    def body(*refs):
        if has_w1:
            g_ref, x_ref, w0_ref, w1_ref, a_ref, out_ref = refs
        else:
            g_ref, x_ref, w0_ref, a_ref, out_ref = refs
        xi = x_ref[...]
        a0 = a_ref[0]
        acc = None
        for j in range(K):
            xj = g_ref[j][:, :c]
            feat = jnp.concatenate([xj - xi, xi], axis=1)
            f1 = _prelu(jnp.dot(feat, w0_ref[...],
                                preferred_element_type=jnp.float32), a0)
            if has_w1:
                f1 = _prelu(jnp.dot(f1, w1_ref[...],
                                    preferred_element_type=jnp.float32), a_ref[1])
            acc = f1 if acc is None else jnp.maximum(acc, f1)
        out_ref[...] = acc

    in_specs = [
        pl.BlockSpec((K, R_MLP, 2 * F), lambda r: (0, r, 0)),
        pl.BlockSpec((R_MLP, c), lambda r: (r, 0)),
        pl.BlockSpec((2 * c, F), lambda r: (0, 0)),
    ]
    args = [G3, Xc, W0]
    if has_w1:
        in_specs.append(pl.BlockSpec((F, F), lambda r: (0, 0)))
        args.append(W1)
    in_specs.append(pl.BlockSpec(memory_space=pltpu.SMEM))
    args.append(alphas)

    return pl.pallas_call(
        body,
        grid=(NB,),
        in_specs=in_specs,
        out_specs=pl.BlockSpec((R_MLP, F), lambda r: (r, 0)),
        out_shape=jax.ShapeDtypeStruct((BN, F), jnp.float32),
    )(*args)


def _edge_stage(x3d, W0, a0, W1, a1):
    B, n, c = x3d.shape
    idxT, tab = _knn_tab(x3d)
    G = _sc_gather(tab, idxT.reshape(-1))
    G3 = G.reshape(K, B * n, 2 * F)
    if W1 is None:
        alphas = a0
    else:
        alphas = jnp.concatenate([a0, a1])
    out = _edge_mlp(G3, x3d.reshape(B * n, c), W0, W1, alphas)
    return out.reshape(B, n, F)


def _global_max(x1, x2, x3, c4w, a):
    B, n, _ = x1.shape
    NB = n // R_MLP

    def body(x1_ref, x2_ref, x3_ref, w_ref, a_ref, m_ref):
        x4 = jnp.concatenate([x1_ref[0], x2_ref[0], x3_ref[0]], axis=1)
        t = _prelu(jnp.dot(x4, w_ref[...], preferred_element_type=jnp.float32),
                   a_ref[0])
        cm = jnp.max(t, axis=0, keepdims=True)

        @pl.when(pl.program_id(1) == 0)
        def _():
            m_ref[0] = cm

        @pl.when(pl.program_id(1) > 0)
        def _():
            m_ref[0] = jnp.maximum(m_ref[0], cm)

    return pl.pallas_call(
        body,
        grid=(B, NB),
        in_specs=[
            pl.BlockSpec((1, R_MLP, F), lambda b, r: (b, r, 0)),
            pl.BlockSpec((1, R_MLP, F), lambda b, r: (b, r, 0)),
            pl.BlockSpec((1, R_MLP, F), lambda b, r: (b, r, 0)),
            pl.BlockSpec((192, 1024), lambda b, r: (0, 0)),
            pl.BlockSpec(memory_space=pltpu.SMEM),
        ],
        out_specs=pl.BlockSpec((1, 1, 1024), lambda b, r: (b, 0, 0)),
        out_shape=jax.ShapeDtypeStruct((B, 1, 1024), jnp.float32),
    )(x1, x2, x3, c4w, a)


def _tail_mlp(x1, x2, x3, m, l1w, l2w, l3w, l4w, alphas):
    B, n, _ = x1.shape
    NB = n // R_MLP

    def body(x1_ref, x2_ref, x3_ref, m_ref, l1_ref, l2_ref, l3_ref, l4_ref,
             a_ref, out_ref):
        x4 = jnp.concatenate([x1_ref[0], x2_ref[0], x3_ref[0]], axis=1)
        mrow = m_ref[0]
        bias = jnp.dot(mrow, l1_ref[192:, :], preferred_element_type=jnp.float32)
        t = _prelu(jnp.dot(x4, l1_ref[0:192, :],
                           preferred_element_type=jnp.float32) + bias, a_ref[0])
        t = _prelu(jnp.dot(t, l2_ref[...], preferred_element_type=jnp.float32),
                   a_ref[1])
        t = _prelu(jnp.dot(t, l3_ref[...], preferred_element_type=jnp.float32),
                   a_ref[2])
        y = _prelu(jnp.dot(t, l4_ref[...], preferred_element_type=jnp.float32),
                   a_ref[3])
        out_ref[0] = y

    return pl.pallas_call(
        body,
        grid=(B, NB),
        in_specs=[
            pl.BlockSpec((1, R_MLP, F), lambda b, r: (b, r, 0)),
            pl.BlockSpec((1, R_MLP, F), lambda b, r: (b, r, 0)),
            pl.BlockSpec((1, R_MLP, F), lambda b, r: (b, r, 0)),
            pl.BlockSpec((1, 1, 1024), lambda b, r: (b, 0, 0)),
            pl.BlockSpec((1216, 256), lambda b, r: (0, 0)),
            pl.BlockSpec((256, 256), lambda b, r: (0, 0)),
            pl.BlockSpec((256, 128), lambda b, r: (0, 0)),
            pl.BlockSpec((128, 4), lambda b, r: (0, 0)),
            pl.BlockSpec(memory_space=pltpu.SMEM),
        ],
        out_specs=pl.BlockSpec((1, R_MLP, 4), lambda b, r: (b, r, 0)),
        out_shape=jax.ShapeDtypeStruct((B, n, 4), jnp.float32),
    )(x1, x2, x3, m, l1w, l2w, l3w, l4w, alphas)


def kernel(d, p, n, sdf_W1, sdf_b1, sdf_W2, sdf_b2, c1w0, c1a0, c1w1, c1a1,
           c2w0, c2a0, c2w1, c2a1, c3w0, c3a0, c4w, c4a, l1w, l1a, l2w, l2a,
           l3w, l3a, l4w, l4a):
    x, sdf_res, sdf_grad, li = _head(d, p, n, sdf_W1, sdf_b1, sdf_W2, sdf_b2)
    x1 = _edge_stage(li, c1w0, c1a0, c1w1, c1a1)
    x2 = _edge_stage(x1, c2w0, c2a0, c2w1, c2a1)
    x3 = _edge_stage(x2, c3w0, c3a0, None, None)
    m = _global_max(x1, x2, x3, c4w, c4a)
    tail_a = jnp.concatenate([l1a, l2a, l3a, l4a])
    y = _tail_mlp(x1, x2, x3, m, l1w, l2w, l3w, l4w, tail_a)
    return (y[..., 0:1], y[..., 1:2], y[..., 2:3], y[..., 3:4],
            x, sdf_res, sdf_grad)


# speedup vs baseline: 16.7182x; 1.0409x over previous
import functools

import jax
import jax.numpy as jnp
from jax import lax
from jax.experimental import pallas as pl
from jax.experimental.pallas import tpu as pltpu
from jax.experimental.pallas import tpu_sc as plsc

K = 10
F = 64
R_KNN = 256
R_MLP = 512
NW = 32
CH = 128


def _prelu(v, a):
    return jnp.where(v >= 0, v, a * v)


def _head(d, p, nrm, W1, b1, W2, b2):
    B, n, _ = d.shape
    W1T = W1.T
    W2T = W2.T

    def body(d_ref, p_ref, n_ref, w1_ref, b1_ref, w2_ref, w2t_ref, w1t_ref,
             b2_ref, x_ref, res_ref, grad_ref, li_ref):
        dd = d_ref[0]
        pp = p_ref[0]
        nn = n_ref[0]
        x = pp + dd * nn
        h = jnp.tanh(jnp.dot(x, w1_ref[...], preferred_element_type=jnp.float32)
                     + b1_ref[...][None, :])
        res = jnp.dot(h, w2_ref[...], preferred_element_type=jnp.float32) + b2_ref[0]
        gp = (1.0 - h * h) * w2t_ref[...]
        grad = jnp.dot(gp, w1t_ref[...], preferred_element_type=jnp.float32)
        x_ref[0] = x
        res_ref[0] = res
        grad_ref[0] = grad
        li_ref[0] = jnp.concatenate([x, nn, res, grad], axis=1)

    return pl.pallas_call(
        body,
        grid=(B,),
        in_specs=[
            pl.BlockSpec((1, n, 1), lambda b: (b, 0, 0)),
            pl.BlockSpec((1, 1, 3), lambda b: (b, 0, 0)),
            pl.BlockSpec((1, n, 3), lambda b: (b, 0, 0)),
            pl.BlockSpec((3, F), lambda b: (0, 0)),
            pl.BlockSpec((F,), lambda b: (0,)),
            pl.BlockSpec((F, 1), lambda b: (0, 0)),
            pl.BlockSpec((1, F), lambda b: (0, 0)),
            pl.BlockSpec((F, 3), lambda b: (0, 0)),
            pl.BlockSpec(memory_space=pltpu.SMEM),
        ],
        out_specs=[
            pl.BlockSpec((1, n, 3), lambda b: (b, 0, 0)),
            pl.BlockSpec((1, n, 1), lambda b: (b, 0, 0)),
            pl.BlockSpec((1, n, 3), lambda b: (b, 0, 0)),
            pl.BlockSpec((1, n, 10), lambda b: (b, 0, 0)),
        ],
        out_shape=[
            jax.ShapeDtypeStruct((B, n, 3), jnp.float32),
            jax.ShapeDtypeStruct((B, n, 1), jnp.float32),
            jax.ShapeDtypeStruct((B, n, 3), jnp.float32),
            jax.ShapeDtypeStruct((B, n, 10), jnp.float32),
        ],
    )(d, p, nrm, W1, b1, W2, W2T, W1T, b2)


def _knn_tab(x3d):
    B, n, c = x3d.shape
    xT = jnp.swapaxes(x3d, 1, 2)
    NB = n // R_KNN

    def body(xf_ref, xt_ref, xb_ref, idx_ref, tab_ref):
        b = pl.program_id(0)
        xf = xf_ref[0]
        xt = xt_ref[0]
        sq_all = jnp.sum(xf * xf, axis=1, keepdims=True)
        sq_r = jnp.sum(xt * xt, axis=0, keepdims=True)
        prod = lax.dot_general(xf, xt, (((1,), (0,)), ((), ())),
                               preferred_element_type=jnp.float32)
        d2 = sq_all + sq_r - 2.0 * prod
        iota = lax.broadcasted_iota(jnp.int32, (n, R_KNN), 0)
        base = b * n
        for t in range(K):
            m = jnp.min(d2, axis=0, keepdims=True)
            amin = jnp.min(jnp.where(d2 == m, iota, n), axis=0, keepdims=True)
            idx_ref[pl.ds(t, 1), :] = amin + base
            d2 = jnp.where(iota == amin, jnp.inf, d2)
        xb = xb_ref[0]
        tab_ref[...] = jnp.concatenate(
            [xb, jnp.zeros((R_KNN, 2 * F - c), jnp.float32)], axis=1)

    return pl.pallas_call(
        body,
        grid=(B, NB),
        in_specs=[
            pl.BlockSpec((1, n, c), lambda b, r: (b, 0, 0)),
            pl.BlockSpec((1, c, R_KNN), lambda b, r: (b, 0, r)),
            pl.BlockSpec((1, R_KNN, c), lambda b, r: (b, r, 0)),
        ],
        out_specs=[
            pl.BlockSpec((K, R_KNN), lambda b, r: (0, b * NB + r)),
            pl.BlockSpec((R_KNN, 2 * F), lambda b, r: (b * NB + r, 0)),
        ],
        out_shape=[
            jax.ShapeDtypeStruct((K, B * n), jnp.int32),
            jax.ShapeDtypeStruct((B * n, 2 * F), jnp.float32),
        ],
    )(x3d, xT, x3d)


def _sc_gather(table, idx):
    nrows = idx.shape[0]
    width = table.shape[1]
    per_w = nrows // NW
    nch = per_w // CH
    mesh = plsc.VectorSubcoreMesh(core_axis_name="c", subcore_axis_name="s")

    @functools.partial(
        pl.kernel,
        out_type=jax.ShapeDtypeStruct((nrows, width), jnp.float32),
        mesh=mesh,
        scratch_types=[
            pltpu.VMEM((per_w,), jnp.int32),
            pltpu.VMEM((CH, width), jnp.float32),
            pltpu.VMEM((CH, width), jnp.float32),
            pltpu.SemaphoreType.DMA,
            pltpu.SemaphoreType.DMA,
            pltpu.SemaphoreType.DMA,
            pltpu.SemaphoreType.DMA,
        ],
    )
    def k(table_hbm, idx_hbm, out_hbm, idx_v, rows_a, rows_b, sga, sgb, swa, swb):
        wid = lax.axis_index("s") * 2 + lax.axis_index("c")
        base = wid * per_w
        rows = [rows_a, rows_b]
        gsem = [sga, sgb]
        wsem = [swa, swb]
        pltpu.sync_copy(idx_hbm.at[pl.ds(base, per_w)], idx_v)
        hg = [None, None]
        hw = [None, None]
        hg[0] = pltpu.async_copy(
            table_hbm.at[idx_v.at[pl.ds(0, CH)]], rows[0], gsem[0])
        for ci in range(nch):
            cur = ci % 2
            nxt = (ci + 1) % 2
            if ci + 1 < nch:
                if hw[nxt] is not None:
                    hw[nxt].wait()
                hg[nxt] = pltpu.async_copy(
                    table_hbm.at[idx_v.at[pl.ds((ci + 1) * CH, CH)]],
                    rows[nxt], gsem[nxt])
            hg[cur].wait()
            hw[cur] = pltpu.async_copy(
                rows[cur], out_hbm.at[pl.ds(base + ci * CH, CH)], wsem[cur])
        hw[(nch - 1) % 2].wait()
        hw[nch % 2].wait()

    return k(table, idx)


def _edge_mlp(G3, Xc, W0, W1, alphas):
    BN, c = Xc.shape
    NB = BN // R_MLP
    has_w1 = W1 is not None

    def body(*refs):
        if has_w1:
            g_ref, x_ref, w0_ref, w1_ref, a_ref, out_ref = refs
        else:
            g_ref, x_ref, w0_ref, a_ref, out_ref = refs
        xi = x_ref[...]
        a0 = a_ref[0]
        acc = None
        for j in range(K):
            xj = g_ref[j][:, :c]
            feat = jnp.concatenate([xj - xi, xi], axis=1)
            f1 = _prelu(jnp.dot(feat, w0_ref[...],
                                preferred_element_type=jnp.float32), a0)
            if has_w1:
                f1 = _prelu(jnp.dot(f1, w1_ref[...],
                                    preferred_element_type=jnp.float32), a_ref[1])
            acc = f1 if acc is None else jnp.maximum(acc, f1)
        out_ref[...] = acc

    in_specs = [
        pl.BlockSpec((K, R_MLP, 2 * F), lambda r: (0, r, 0)),
        pl.BlockSpec((R_MLP, c), lambda r: (r, 0)),
        pl.BlockSpec((2 * c, F), lambda r: (0, 0)),
    ]
    args = [G3, Xc, W0]
    if has_w1:
        in_specs.append(pl.BlockSpec((F, F), lambda r: (0, 0)))
        args.append(W1)
    in_specs.append(pl.BlockSpec(memory_space=pltpu.SMEM))
    args.append(alphas)

    return pl.pallas_call(
        body,
        grid=(NB,),
        in_specs=in_specs,
        out_specs=pl.BlockSpec((R_MLP, F), lambda r: (r, 0)),
        out_shape=jax.ShapeDtypeStruct((BN, F), jnp.float32),
    )(*args)


def _edge_stage(x3d, W0, a0, W1, a1):
    B, n, c = x3d.shape
    idxT, tab = _knn_tab(x3d)
    G = _sc_gather(tab, idxT.reshape(-1))
    G3 = G.reshape(K, B * n, 2 * F)
    if W1 is None:
        alphas = a0
    else:
        alphas = jnp.concatenate([a0, a1])
    out = _edge_mlp(G3, x3d.reshape(B * n, c), W0, W1, alphas)
    return out.reshape(B, n, F)


def _global_max(x1, x2, x3, c4w, a):
    B, n, _ = x1.shape
    NB = n // R_MLP

    def body(x1_ref, x2_ref, x3_ref, w_ref, a_ref, m_ref):
        x4 = jnp.concatenate([x1_ref[0], x2_ref[0], x3_ref[0]], axis=1)
        t = _prelu(jnp.dot(x4, w_ref[...], preferred_element_type=jnp.float32),
                   a_ref[0])
        cm = jnp.max(t, axis=0, keepdims=True)

        @pl.when(pl.program_id(1) == 0)
        def _():
            m_ref[0] = cm

        @pl.when(pl.program_id(1) > 0)
        def _():
            m_ref[0] = jnp.maximum(m_ref[0], cm)

    return pl.pallas_call(
        body,
        grid=(B, NB),
        in_specs=[
            pl.BlockSpec((1, R_MLP, F), lambda b, r: (b, r, 0)),
            pl.BlockSpec((1, R_MLP, F), lambda b, r: (b, r, 0)),
            pl.BlockSpec((1, R_MLP, F), lambda b, r: (b, r, 0)),
            pl.BlockSpec((192, 1024), lambda b, r: (0, 0)),
            pl.BlockSpec(memory_space=pltpu.SMEM),
        ],
        out_specs=pl.BlockSpec((1, 1, 1024), lambda b, r: (b, 0, 0)),
        out_shape=jax.ShapeDtypeStruct((B, 1, 1024), jnp.float32),
    )(x1, x2, x3, c4w, a)


def _tail_mlp(x1, x2, x3, m, l1w, l2w, l3w, l4w, alphas):
    B, n, _ = x1.shape
    NB = n // R_MLP

    def body(x1_ref, x2_ref, x3_ref, m_ref, l1_ref, l2_ref, l3_ref, l4_ref,
             a_ref, out_ref):
        x4 = jnp.concatenate([x1_ref[0], x2_ref[0], x3_ref[0]], axis=1)
        mrow = m_ref[0]
        bias = jnp.dot(mrow, l1_ref[192:, :], preferred_element_type=jnp.float32)
        t = _prelu(jnp.dot(x4, l1_ref[0:192, :],
                           preferred_element_type=jnp.float32) + bias, a_ref[0])
        t = _prelu(jnp.dot(t, l2_ref[...], preferred_element_type=jnp.float32),
                   a_ref[1])
        t = _prelu(jnp.dot(t, l3_ref[...], preferred_element_type=jnp.float32),
                   a_ref[2])
        y = _prelu(jnp.dot(t, l4_ref[...], preferred_element_type=jnp.float32),
                   a_ref[3])
        out_ref[0] = y

    return pl.pallas_call(
        body,
        grid=(B, NB),
        in_specs=[
            pl.BlockSpec((1, R_MLP, F), lambda b, r: (b, r, 0)),
            pl.BlockSpec((1, R_MLP, F), lambda b, r: (b, r, 0)),
            pl.BlockSpec((1, R_MLP, F), lambda b, r: (b, r, 0)),
            pl.BlockSpec((1, 1, 1024), lambda b, r: (b, 0, 0)),
            pl.BlockSpec((1216, 256), lambda b, r: (0, 0)),
            pl.BlockSpec((256, 256), lambda b, r: (0, 0)),
            pl.BlockSpec((256, 128), lambda b, r: (0, 0)),
            pl.BlockSpec((128, 4), lambda b, r: (0, 0)),
            pl.BlockSpec(memory_space=pltpu.SMEM),
        ],
        out_specs=pl.BlockSpec((1, R_MLP, 4), lambda b, r: (b, r, 0)),
        out_shape=jax.ShapeDtypeStruct((B, n, 4), jnp.float32),
    )(x1, x2, x3, m, l1w, l2w, l3w, l4w, alphas)


def kernel(d, p, n, sdf_W1, sdf_b1, sdf_W2, sdf_b2, c1w0, c1a0, c1w1, c1a1,
           c2w0, c2a0, c2w1, c2a1, c3w0, c3a0, c4w, c4a, l1w, l1a, l2w, l2a,
           l3w, l3a, l4w, l4a):
    x, sdf_res, sdf_grad, li = _head(d, p, n, sdf_W1, sdf_b1, sdf_W2, sdf_b2)
    x1 = _edge_stage(li, c1w0, c1a0, c1w1, c1a1)
    x2 = _edge_stage(x1, c2w0, c2a0, c2w1, c2a1)
    x3 = _edge_stage(x2, c3w0, c3a0, None, None)
    m = _global_max(x1, x2, x3, c4w, c4a)
    tail_a = jnp.concatenate([l1a, l2a, l3a, l4a])
    y = _tail_mlp(x1, x2, x3, m, l1w, l2w, l3w, l4w, tail_a)
    return (y[..., 0:1], y[..., 1:2], y[..., 2:3], y[..., 3:4],
            x, sdf_res, sdf_grad)


# speedup vs baseline: 25.2489x; 1.5103x over previous
import functools

import jax
import jax.numpy as jnp
from jax import lax
from jax.experimental import pallas as pl
from jax.experimental.pallas import tpu as pltpu
from jax.experimental.pallas import tpu_sc as plsc

K = 10
F = 64
R_KNN = 256
R_MLP = 512
NW = 32
CH = 128


def _prelu(v, a):
    return jnp.where(v >= 0, v, a * v)


def _head(d, p, nrm, W1, b1, W2, b2):
    B, n, _ = d.shape
    W1T = W1.T
    W2T = W2.T

    def body(d_ref, p_ref, n_ref, w1_ref, b1_ref, w2_ref, w2t_ref, w1t_ref,
             b2_ref, x_ref, res_ref, grad_ref, li_ref):
        dd = d_ref[0]
        pp = p_ref[0]
        nn = n_ref[0]
        x = pp + dd * nn
        h = jnp.tanh(jnp.dot(x, w1_ref[...], preferred_element_type=jnp.float32)
                     + b1_ref[...][None, :])
        res = jnp.dot(h, w2_ref[...], preferred_element_type=jnp.float32) + b2_ref[0]
        gp = (1.0 - h * h) * w2t_ref[...]
        grad = jnp.dot(gp, w1t_ref[...], preferred_element_type=jnp.float32)
        x_ref[0] = x
        res_ref[0] = res
        grad_ref[0] = grad
        li_ref[0] = jnp.concatenate([x, nn, res, grad], axis=1)

    return pl.pallas_call(
        body,
        grid=(B,),
        in_specs=[
            pl.BlockSpec((1, n, 1), lambda b: (b, 0, 0)),
            pl.BlockSpec((1, 1, 3), lambda b: (b, 0, 0)),
            pl.BlockSpec((1, n, 3), lambda b: (b, 0, 0)),
            pl.BlockSpec((3, F), lambda b: (0, 0)),
            pl.BlockSpec((F,), lambda b: (0,)),
            pl.BlockSpec((F, 1), lambda b: (0, 0)),
            pl.BlockSpec((1, F), lambda b: (0, 0)),
            pl.BlockSpec((F, 3), lambda b: (0, 0)),
            pl.BlockSpec(memory_space=pltpu.SMEM),
        ],
        out_specs=[
            pl.BlockSpec((1, n, 3), lambda b: (b, 0, 0)),
            pl.BlockSpec((1, n, 1), lambda b: (b, 0, 0)),
            pl.BlockSpec((1, n, 3), lambda b: (b, 0, 0)),
            pl.BlockSpec((1, n, 10), lambda b: (b, 0, 0)),
        ],
        out_shape=[
            jax.ShapeDtypeStruct((B, n, 3), jnp.float32),
            jax.ShapeDtypeStruct((B, n, 1), jnp.float32),
            jax.ShapeDtypeStruct((B, n, 3), jnp.float32),
            jax.ShapeDtypeStruct((B, n, 10), jnp.float32),
        ],
    )(d, p, nrm, W1, b1, W2, W2T, W1T, b2)


def _knn_tab(x3d):
    B, n, c = x3d.shape
    xT = jnp.swapaxes(x3d, 1, 2)
    NB = n // R_KNN

    def body(xf_ref, xt_ref, xb_ref, idx_ref, tab_ref):
        b = pl.program_id(0)
        xf = xf_ref[0]
        xt = xt_ref[0]
        sq_all = jnp.sum(xf * xf, axis=1, keepdims=True)
        sq_r = jnp.sum(xt * xt, axis=0, keepdims=True)
        prod = lax.dot_general(xf, xt, (((1,), (0,)), ((), ())),
                               preferred_element_type=jnp.float32)
        d2 = sq_all + sq_r - 2.0 * prod
        iota = lax.broadcasted_iota(jnp.int32, (n, R_KNN), 0)
        base = b * n
        u = lax.bitcast_convert_type(d2, jnp.int32)
        key = u ^ ((u >> 31) & jnp.int32(0x7FFFFFFF))
        packed = (key & jnp.int32(-4096)) | iota
        prev = jnp.full((1, R_KNN), jnp.iinfo(jnp.int32).min, jnp.int32)
        big = jnp.int32(jnp.iinfo(jnp.int32).max)
        for t in range(K):
            cur = jnp.min(jnp.where(packed > prev, packed, big),
                          axis=0, keepdims=True)
            idx_ref[pl.ds(t, 1), :] = (cur & jnp.int32(4095)) + base
            prev = cur
        xb = xb_ref[0]
        tab_ref[...] = jnp.concatenate(
            [xb, jnp.zeros((R_KNN, 2 * F - c), jnp.float32)], axis=1)

    return pl.pallas_call(
        body,
        grid=(B, NB),
        in_specs=[
            pl.BlockSpec((1, n, c), lambda b, r: (b, 0, 0)),
            pl.BlockSpec((1, c, R_KNN), lambda b, r: (b, 0, r)),
            pl.BlockSpec((1, R_KNN, c), lambda b, r: (b, r, 0)),
        ],
        out_specs=[
            pl.BlockSpec((K, R_KNN), lambda b, r: (0, b * NB + r)),
            pl.BlockSpec((R_KNN, 2 * F), lambda b, r: (b * NB + r, 0)),
        ],
        out_shape=[
            jax.ShapeDtypeStruct((K, B * n), jnp.int32),
            jax.ShapeDtypeStruct((B * n, 2 * F), jnp.float32),
        ],
    )(x3d, xT, x3d)


def _sc_gather(table, idx):
    nrows = idx.shape[0]
    width = table.shape[1]
    per_w = nrows // NW
    nch = per_w // CH
    mesh = plsc.VectorSubcoreMesh(core_axis_name="c", subcore_axis_name="s")

    @functools.partial(
        pl.kernel,
        out_type=jax.ShapeDtypeStruct((nrows, width), jnp.float32),
        mesh=mesh,
        scratch_types=[
            pltpu.VMEM((per_w,), jnp.int32),
            pltpu.VMEM((CH, width), jnp.float32),
            pltpu.VMEM((CH, width), jnp.float32),
            pltpu.SemaphoreType.DMA,
            pltpu.SemaphoreType.DMA,
            pltpu.SemaphoreType.DMA,
            pltpu.SemaphoreType.DMA,
        ],
    )
    def k(table_hbm, idx_hbm, out_hbm, idx_v, rows_a, rows_b, sga, sgb, swa, swb):
        wid = lax.axis_index("s") * 2 + lax.axis_index("c")
        base = wid * per_w
        rows = [rows_a, rows_b]
        gsem = [sga, sgb]
        wsem = [swa, swb]
        pltpu.sync_copy(idx_hbm.at[pl.ds(base, per_w)], idx_v)
        hg = [None, None]
        hw = [None, None]
        hg[0] = pltpu.async_copy(
            table_hbm.at[idx_v.at[pl.ds(0, CH)]], rows[0], gsem[0])
        for ci in range(nch):
            cur = ci % 2
            nxt = (ci + 1) % 2
            if ci + 1 < nch:
                if hw[nxt] is not None:
                    hw[nxt].wait()
                hg[nxt] = pltpu.async_copy(
                    table_hbm.at[idx_v.at[pl.ds((ci + 1) * CH, CH)]],
                    rows[nxt], gsem[nxt])
            hg[cur].wait()
            hw[cur] = pltpu.async_copy(
                rows[cur], out_hbm.at[pl.ds(base + ci * CH, CH)], wsem[cur])
        hw[(nch - 1) % 2].wait()
        hw[nch % 2].wait()

    return k(table, idx)


def _edge_mlp(G3, Xc, W0, W1, alphas):
    BN, c = Xc.shape
    NB = BN // R_MLP
    has_w1 = W1 is not None

    def body(*refs):
        if has_w1:
            g_ref, x_ref, w0_ref, w1_ref, a_ref, out_ref = refs
        else:
            g_ref, x_ref, w0_ref, a_ref, out_ref = refs
        xi = x_ref[...]
        a0 = a_ref[0]
        acc = None
        for j in range(K):
            xj = g_ref[j][:, :c]
            feat = jnp.concatenate([xj - xi, xi], axis=1)
            f1 = _prelu(jnp.dot(feat, w0_ref[...],
                                preferred_element_type=jnp.float32), a0)
            if has_w1:
                f1 = _prelu(jnp.dot(f1, w1_ref[...],
                                    preferred_element_type=jnp.float32), a_ref[1])
            acc = f1 if acc is None else jnp.maximum(acc, f1)
        out_ref[...] = acc

    in_specs = [
        pl.BlockSpec((K, R_MLP, 2 * F), lambda r: (0, r, 0)),
        pl.BlockSpec((R_MLP, c), lambda r: (r, 0)),
        pl.BlockSpec((2 * c, F), lambda r: (0, 0)),
    ]
    args = [G3, Xc, W0]
    if has_w1:
        in_specs.append(pl.BlockSpec((F, F), lambda r: (0, 0)))
        args.append(W1)
    in_specs.append(pl.BlockSpec(memory_space=pltpu.SMEM))
    args.append(alphas)

    return pl.pallas_call(
        body,
        grid=(NB,),
        in_specs=in_specs,
        out_specs=pl.BlockSpec((R_MLP, F), lambda r: (r, 0)),
        out_shape=jax.ShapeDtypeStruct((BN, F), jnp.float32),
    )(*args)


def _edge_stage(x3d, W0, a0, W1, a1):
    B, n, c = x3d.shape
    idxT, tab = _knn_tab(x3d)
    G = _sc_gather(tab, idxT.reshape(-1))
    G3 = G.reshape(K, B * n, 2 * F)
    if W1 is None:
        alphas = a0
    else:
        alphas = jnp.concatenate([a0, a1])
    out = _edge_mlp(G3, x3d.reshape(B * n, c), W0, W1, alphas)
    return out.reshape(B, n, F)


def _global_max(x1, x2, x3, c4w, a):
    B, n, _ = x1.shape
    NB = n // R_MLP

    def body(x1_ref, x2_ref, x3_ref, w_ref, a_ref, m_ref):
        x4 = jnp.concatenate([x1_ref[0], x2_ref[0], x3_ref[0]], axis=1)
        t = _prelu(jnp.dot(x4, w_ref[...], preferred_element_type=jnp.float32),
                   a_ref[0])
        cm = jnp.max(t, axis=0, keepdims=True)

        @pl.when(pl.program_id(1) == 0)
        def _():
            m_ref[0] = cm

        @pl.when(pl.program_id(1) > 0)
        def _():
            m_ref[0] = jnp.maximum(m_ref[0], cm)

    return pl.pallas_call(
        body,
        grid=(B, NB),
        in_specs=[
            pl.BlockSpec((1, R_MLP, F), lambda b, r: (b, r, 0)),
            pl.BlockSpec((1, R_MLP, F), lambda b, r: (b, r, 0)),
            pl.BlockSpec((1, R_MLP, F), lambda b, r: (b, r, 0)),
            pl.BlockSpec((192, 1024), lambda b, r: (0, 0)),
            pl.BlockSpec(memory_space=pltpu.SMEM),
        ],
        out_specs=pl.BlockSpec((1, 1, 1024), lambda b, r: (b, 0, 0)),
        out_shape=jax.ShapeDtypeStruct((B, 1, 1024), jnp.float32),
    )(x1, x2, x3, c4w, a)


def _tail_mlp(x1, x2, x3, m, l1w, l2w, l3w, l4w, alphas):
    B, n, _ = x1.shape
    NB = n // R_MLP

    def body(x1_ref, x2_ref, x3_ref, m_ref, l1_ref, l2_ref, l3_ref, l4_ref,
             a_ref, out_ref):
        x4 = jnp.concatenate([x1_ref[0], x2_ref[0], x3_ref[0]], axis=1)
        mrow = m_ref[0]
        bias = jnp.dot(mrow, l1_ref[192:, :], preferred_element_type=jnp.float32)
        t = _prelu(jnp.dot(x4, l1_ref[0:192, :],
                           preferred_element_type=jnp.float32) + bias, a_ref[0])
        t = _prelu(jnp.dot(t, l2_ref[...], preferred_element_type=jnp.float32),
                   a_ref[1])
        t = _prelu(jnp.dot(t, l3_ref[...], preferred_element_type=jnp.float32),
                   a_ref[2])
        y = _prelu(jnp.dot(t, l4_ref[...], preferred_element_type=jnp.float32),
                   a_ref[3])
        out_ref[0] = y

    return pl.pallas_call(
        body,
        grid=(B, NB),
        in_specs=[
            pl.BlockSpec((1, R_MLP, F), lambda b, r: (b, r, 0)),
            pl.BlockSpec((1, R_MLP, F), lambda b, r: (b, r, 0)),
            pl.BlockSpec((1, R_MLP, F), lambda b, r: (b, r, 0)),
            pl.BlockSpec((1, 1, 1024), lambda b, r: (b, 0, 0)),
            pl.BlockSpec((1216, 256), lambda b, r: (0, 0)),
            pl.BlockSpec((256, 256), lambda b, r: (0, 0)),
            pl.BlockSpec((256, 128), lambda b, r: (0, 0)),
            pl.BlockSpec((128, 4), lambda b, r: (0, 0)),
            pl.BlockSpec(memory_space=pltpu.SMEM),
        ],
        out_specs=pl.BlockSpec((1, R_MLP, 4), lambda b, r: (b, r, 0)),
        out_shape=jax.ShapeDtypeStruct((B, n, 4), jnp.float32),
    )(x1, x2, x3, m, l1w, l2w, l3w, l4w, alphas)


def kernel(d, p, n, sdf_W1, sdf_b1, sdf_W2, sdf_b2, c1w0, c1a0, c1w1, c1a1,
           c2w0, c2a0, c2w1, c2a1, c3w0, c3a0, c4w, c4a, l1w, l1a, l2w, l2a,
           l3w, l3a, l4w, l4a):
    x, sdf_res, sdf_grad, li = _head(d, p, n, sdf_W1, sdf_b1, sdf_W2, sdf_b2)
    x1 = _edge_stage(li, c1w0, c1a0, c1w1, c1a1)
    x2 = _edge_stage(x1, c2w0, c2a0, c2w1, c2a1)
    x3 = _edge_stage(x2, c3w0, c3a0, None, None)
    m = _global_max(x1, x2, x3, c4w, c4a)
    tail_a = jnp.concatenate([l1a, l2a, l3a, l4a])
    y = _tail_mlp(x1, x2, x3, m, l1w, l2w, l3w, l4w, tail_a)
    return (y[..., 0:1], y[..., 1:2], y[..., 2:3], y[..., 3:4],
            x, sdf_res, sdf_grad)
